# baseline (speedup 1.0000x reference)
import functools

import jax
import jax.numpy as jnp
from jax import lax
from jax.experimental import pallas as pl
from jax.experimental.pallas import tpu as pltpu
from jax.experimental.pallas import tpu_sc as plsc

_MAX_K = 128
_SEQ = 2048
_D = 64
_T_ROWS = 2 * _MAX_K + 1
_Q_PER_TILE = _SEQ // 32
_HALF_W = (_SEQ // 2) * _D
_WIN = 1024 + _Q_PER_TILE
_MARG = _T_ROWS
_EXT = _WIN + 2 * _MARG
_RING = 8


def _sc_body(w_hbm, out_hbm, wbuf, wext, sem):
    c = lax.axis_index("c")
    s = lax.axis_index("s")
    wid = s * 2 + c
    q0 = wid * _Q_PER_TILE

    pltpu.sync_copy(w_hbm.at[pl.ds(0, _D)], wbuf.at[pl.ds(0, _D)])
    pltpu.sync_copy(w_hbm.at[pl.ds(256 * _D, _D)], wbuf.at[pl.ds(_D, _D)])
    c0 = [wbuf[pl.ds(j * 16, 16)] for j in range(4)]
    cz = [wbuf[pl.ds(_D + j * 16, 16)] for j in range(4)]

    def _drain_one():
        pltpu.make_async_copy(wext.at[pl.ds(0, _HALF_W)],
                              out_hbm.at[pl.ds(0, _HALF_W)], sem).wait()

    for h in (0, 1):
        lo = 1024 * h + _SEQ - 1 - (q0 + _Q_PER_TILE - 1)
        p = 1919 - lo
        a = jnp.clip(p, 0, _WIN)
        b = jnp.clip(p + _T_ROWS, 0, _WIN)
        pc = jnp.clip(p, -_MARG, _WIN + _MARG - _T_ROWS)

        def fill(vj):
            def body(i, _):
                for j in range(4):
                    wext[pl.ds((_MARG + i) * _D + j * 16, 16)] = vj[j]
                return 0
            return body

        lax.fori_loop(0, a, fill(c0), 0)
        lax.fori_loop(b, _WIN, fill(cz), 0)
        pltpu.sync_copy(w_hbm,
                        wext.at[pl.ds((_MARG + pc) * _D, _T_ROWS * _D)])

        def _start(k):
            src = wext.at[pl.ds((_MARG + _Q_PER_TILE - 1 - k) * _D, _HALF_W)]
            dst = out_hbm.at[pl.ds((2 * (q0 + k) + h) * _HALF_W, _HALF_W)]
            pltpu.async_copy(src, dst, sem)

        for j in range(_RING):
            _start(j)

        def _steady(k, _):
            _drain_one()
            _start(_RING + k)
            return 0

        lax.fori_loop(0, _Q_PER_TILE - _RING, _steady, 0)
        for j in range(_RING):
            _drain_one()


def kernel(seq_len, emb_weight):
    del seq_len
    mesh = plsc.VectorSubcoreMesh(core_axis_name="c", subcore_axis_name="s")
    run = functools.partial(
        pl.kernel,
        mesh=mesh,
        out_type=jax.ShapeDtypeStruct((_SEQ * _SEQ * _D,), jnp.float32),
        scratch_types=[
            pltpu.VMEM((2 * _D,), jnp.float32),
            pltpu.VMEM((_EXT * _D,), jnp.float32),
            pltpu.SemaphoreType.DMA,
        ],
    )(_sc_body)
    out = run(emb_weight.reshape(-1))
    return out.reshape(_SEQ, _SEQ, _D)


# speedup vs baseline: 1.2750x; 1.2750x over previous
import functools

import jax
import jax.numpy as jnp
from jax import lax
from jax.experimental import pallas as pl
from jax.experimental.pallas import tpu as pltpu
from jax.experimental.pallas import tpu_sc as plsc

_MAX_K = 128
_SEQ = 2048
_D = 64
_T_ROWS = 2 * _MAX_K + 1
_Q_PER_TILE = _SEQ // 32
_QCOL = _SEQ // 4
_WIN = _QCOL + _Q_PER_TILE
_RING = 8


def _sc_body(w_hbm, out_hbm, tbuf, wext, sem):
    c = lax.axis_index("c")
    s = lax.axis_index("s")
    wid = s * 2 + c
    q0 = wid * _Q_PER_TILE

    pltpu.sync_copy(w_hbm, tbuf)
    c0 = [tbuf[pl.ds(j * 16, 16)] for j in range(4)]
    cz = [tbuf[pl.ds(256 * _D + j * 16, 16)] for j in range(4)]

    def _drain_one():
        pltpu.make_async_copy(wext.at[pl.ds(0, _QCOL), :],
                              out_hbm.at[0, pl.ds(0, _QCOL), :], sem).wait()

    for h4 in range(4):
        lo = _QCOL * h4 + _SEQ - 1 - (q0 + _Q_PER_TILE - 1)
        p = 1919 - lo
        a = jnp.clip(p, 0, _WIN)
        b = jnp.clip(p + _T_ROWS, 0, _WIN)

        def fill(vj):
            def body(i, _):
                for j in range(4):
                    wext[i, pl.ds(j * 16, 16)] = vj[j]
                return 0
            return body

        def spread(r, _):
            for j in range(4):
                wext[r, pl.ds(j * 16, 16)] = tbuf[pl.ds((r - p) * _D + j * 16,
                                                        16)]
            return 0

        lax.fori_loop(0, a, fill(c0), 0)
        lax.fori_loop(b, _WIN, fill(cz), 0)
        lax.fori_loop(a, b, spread, 0)

        def _start(k):
            src = wext.at[pl.ds(_Q_PER_TILE - 1 - k, _QCOL), :]
            dst = out_hbm.at[q0 + k, pl.ds(_QCOL * h4, _QCOL), :]
            pltpu.async_copy(src, dst, sem)

        for j in range(_RING):
            _start(j)

        def _steady(k, _):
            _drain_one()
            _start(_RING + k)
            return 0

        lax.fori_loop(0, _Q_PER_TILE - _RING, _steady, 0)
        for j in range(_RING):
            _drain_one()


def kernel(seq_len, emb_weight):
    del seq_len
    mesh = plsc.VectorSubcoreMesh(core_axis_name="c", subcore_axis_name="s")
    run = functools.partial(
        pl.kernel,
        mesh=mesh,
        out_type=jax.ShapeDtypeStruct((_SEQ, _SEQ, _D), jnp.float32),
        scratch_types=[
            pltpu.VMEM((_T_ROWS * _D,), jnp.float32),
            pltpu.VMEM((_WIN, _D), jnp.float32),
            pltpu.SemaphoreType.DMA,
        ],
        compiler_params=pltpu.CompilerParams(use_tc_tiling_on_sc=True),
    )(_sc_body)
    return run(emb_weight.reshape(-1))
